# baseline (speedup 1.0000x reference)
import functools

import jax
import jax.numpy as jnp
from jax import lax
from jax.experimental import pallas as pl
from jax.experimental.pallas import tpu as pltpu
from jax.experimental.pallas import tpu_sc as plsc

D_MODEL = 128
MAX_LEN = 200
VOCAB = 1000
START_TOKEN = 997
LANES = 16
NBUF = 3


def _positional_encoding():
    even_i = jnp.arange(0, D_MODEL, 2).astype(jnp.float32)
    denominator = jnp.power(10000.0, even_i / D_MODEL)
    position = jnp.arange(MAX_LEN, dtype=jnp.float32).reshape(MAX_LEN, 1)
    even_pe = jnp.sin(position / denominator)
    odd_pe = jnp.cos(position / denominator)
    return jnp.stack([even_pe, odd_pe], axis=2).reshape(MAX_LEN, D_MODEL)


@functools.lru_cache(maxsize=None)
def _make_sc_kernel(B: int, V: int = VOCAB):
    info = plsc.get_sparse_core_info()
    NC, NS = info.num_cores, info.num_subcores
    NW = NC * NS
    assert B % NW == 0
    s_per_w = B // NW
    n_outer = -(-(s_per_w + 1) // NBUF)

    mesh = plsc.VectorSubcoreMesh(core_axis_name="c", subcore_axis_name="s")

    @functools.partial(
        pl.kernel,
        mesh=mesh,
        out_type=jax.ShapeDtypeStruct((B * MAX_LEN, D_MODEL), jnp.float32),
        scratch_types=[
            pltpu.VMEM((MAX_LEN, D_MODEL), jnp.float32),
            pltpu.VMEM((MAX_LEN, D_MODEL), jnp.float32),
            pltpu.VMEM((MAX_LEN, D_MODEL), jnp.float32),
            pltpu.VMEM((MAX_LEN, D_MODEL), jnp.float32),
            pltpu.VMEM((MAX_LEN,), jnp.int32),
            pltpu.VMEM((MAX_LEN,), jnp.int32),
            pltpu.VMEM((MAX_LEN,), jnp.int32),
            pltpu.VMEM_SHARED((V, D_MODEL), jnp.float32),
            pltpu.SemaphoreType.DMA,
            pltpu.SemaphoreType.DMA,
            pltpu.SemaphoreType.DMA,
            pltpu.SemaphoreType.DMA,
            pltpu.SemaphoreType.DMA,
            pltpu.SemaphoreType.DMA,
            pltpu.SemaphoreType.DMA,
            pltpu.SemaphoreType.DMA,
            pltpu.SemaphoreType.DMA,
        ],
    )
    def sc_embed(tok_hbm, table_hbm, pe_hbm, out_hbm,
                 pe_v, buf0, buf1, buf2, ib0, ib1, ib2, tab_s,
                 sg0, sg1, sg2, so0, so1, so2, si0, si1, si2):
        wid = lax.axis_index("s") * NC + lax.axis_index("c")
        wbase = wid * s_per_w * MAX_LEN
        bufs = (buf0, buf1, buf2)
        ibufs = (ib0, ib1, ib2)
        gsems = (sg0, sg1, sg2)
        osems = (so0, so1, so2)
        isems = (si0, si1, si2)

        @pl.when(lax.axis_index("s") == 0)
        def _():
            pltpu.sync_copy(table_hbm, tab_s)
        pltpu.sync_copy(pe_hbm, pe_v)
        plsc.subcore_barrier()

        def idx_fetch(j, k):
            pltpu.async_copy(
                tok_hbm.at[pl.ds(wbase + j * MAX_LEN, MAX_LEN)], ibufs[k], isems[k])

        def idx_wait(k):
            pltpu.make_async_copy(
                tok_hbm.at[pl.ds(wbase, MAX_LEN)], ibufs[k], isems[k]).wait()

        def gather(k):
            pltpu.async_copy(tab_s.at[ibufs[k]], bufs[k], gsems[k])

        def gather_wait(k):
            pltpu.make_async_copy(tab_s.at[ibufs[k]], bufs[k], gsems[k]).wait()

        def out_copy(j, k):
            pltpu.async_copy(
                bufs[k], out_hbm.at[pl.ds(wbase + j * MAX_LEN, MAX_LEN)], osems[k])

        def out_wait(k):
            pltpu.make_async_copy(
                bufs[k], out_hbm.at[pl.ds(wbase, MAX_LEN)], osems[k]).wait()

        def add_pe(k):
            def add_rows(t4, c2):
                for dt in range(4):
                    t = t4 * 4 + dt
                    for c in range(D_MODEL // LANES):
                        sl = pl.ds(c * LANES, LANES)
                        plsc.addupdate(bufs[k].at[t, sl], pe_v[t, sl])
                return c2
            lax.fori_loop(0, MAX_LEN // 4, add_rows, 0)

        idx_fetch(0, 0)
        idx_fetch(1, 1)
        idx_fetch(2, 2)
        idx_wait(0)
        gather(0)
        idx_wait(1)
        gather(1)

        def body(g, carry):
            for b in range(NBUF):
                j = NBUF * g + b

                @pl.when(j < s_per_w)
                def _():
                    gather_wait(b)

                @pl.when(j + 3 < s_per_w)
                def _():
                    idx_fetch(j + 3, b)

                @pl.when(j < s_per_w)
                def _():
                    add_pe(b)

                @pl.when((j >= 1) & (j < s_per_w + 1))
                def _():
                    out_wait((b + 2) % NBUF)

                @pl.when(j + 2 < s_per_w)
                def _():
                    idx_wait((b + 2) % NBUF)
                    gather((b + 2) % NBUF)

                @pl.when(j < s_per_w)
                def _():
                    out_copy(j, b)
            return carry

        lax.fori_loop(0, n_outer, body, 0)

    return sc_embed


def kernel(x, table):
    B = x.shape[0]
    start_col = jnp.full((B, 1), START_TOKEN, dtype=x.dtype)
    tok = jnp.concatenate([start_col, x[:, : MAX_LEN - 1]], axis=1).reshape(-1)
    pe = _positional_encoding()
    out = _make_sc_kernel(B, table.shape[0])(tok, table, pe)
    return out.reshape(B, MAX_LEN, D_MODEL)
